# Initial kernel scaffold; baseline (speedup 1.0000x reference)
import jax
import jax.numpy as jnp
from jax.experimental import pallas as pl


def kernel(input_ids, bbox, token_type_ids, word_emb, pos_emb, x_emb, y_emb, h_emb, w_emb, tt_emb, ln_gamma, ln_beta):
    raise NotImplementedError("write your pallas kernel here")


# speedup vs baseline: 1.7760x; 1.7760x over previous
import functools

import jax
import jax.numpy as jnp
from jax import lax
from jax.experimental import pallas as pl
from jax.experimental.pallas import tpu as pltpu
from jax.experimental.pallas import tpu_sc as plsc

_EPS = 1e-12
_NC, _NS = 2, 16
_NW = _NC * _NS
_LANES = 16


def _sc_gather_sum(word_emb, x_emb, y_emb, h_emb, w_emb, ids_flat, bbox_t):
    tok = ids_flat.shape[0]
    hdim = word_emb.shape[1]
    b_per_w = tok // _NW
    ch = 64
    n_chunks = b_per_w // ch
    assert tok % _NW == 0 and b_per_w % ch == 0 and hdim % _LANES == 0

    mesh = plsc.VectorSubcoreMesh(
        core_axis_name="c", subcore_axis_name="s",
        num_cores=_NC, num_subcores=_NS)

    @functools.partial(
        pl.kernel,
        out_type=jax.ShapeDtypeStruct((tok, hdim), jnp.float32),
        mesh=mesh,
        scratch_types=[
            pltpu.VMEM((ch,), jnp.int32),
            pltpu.VMEM((4, ch), jnp.int32),
            pltpu.VMEM((2, ch), jnp.int32),
            pltpu.VMEM((ch, hdim), jnp.float32),
            pltpu.VMEM((ch, hdim), jnp.float32),
            pltpu.SemaphoreType.DMA,
        ],
    )
    def k(word_hbm, x_hbm, y_hbm, h_hbm, w_hbm, ids_hbm, bbox_hbm, out_hbm,
          widx, bidx, hwidx, acc, buf, sem):
        wid = lax.axis_index("s") * _NC + lax.axis_index("c")

        @pl.loop(0, n_chunks)
        def _chunk(c):
            base = wid * b_per_w + c * ch
            pltpu.sync_copy(ids_hbm.at[pl.ds(base, ch)], widx)
            for j in range(4):
                pltpu.sync_copy(bbox_hbm.at[j, pl.ds(base, ch)], bidx.at[j])
            for i in range(0, ch, _LANES):
                s = pl.ds(i, _LANES)
                hwidx[0, s] = bidx[3, s] - bidx[1, s]
                hwidx[1, s] = bidx[2, s] - bidx[0, s]

            pltpu.async_copy(word_hbm.at[widx], acc, sem).wait()
            gathers = (
                (x_hbm, bidx.at[0]), (y_hbm, bidx.at[1]),
                (x_hbm, bidx.at[2]), (y_hbm, bidx.at[3]),
                (h_hbm, hwidx.at[0]), (w_hbm, hwidx.at[1]),
            )
            for tab, iref in gathers:
                pltpu.async_copy(tab.at[iref], buf, sem).wait()

                @pl.loop(0, ch)
                def _row(r):
                    for i in range(0, hdim, _LANES):
                        s = pl.ds(i, _LANES)
                        plsc.addupdate(acc.at[r, s], buf[r, s])

            pltpu.sync_copy(acc, out_hbm.at[pl.ds(base, ch)])

    return k(word_emb, x_emb, y_emb, h_emb, w_emb, ids_flat, bbox_t)


def _tc_finish(gsum, pos_emb, tids2, tt_pad, gamma2, beta2):
    tok, hdim = gsum.shape
    blk = 256
    n = tok // blk
    s_len = pos_emb.shape[0]
    pos_blocks = s_len // blk

    def body(g_ref, pos_ref, tid_ref, ttab_ref, gam_ref, bet_ref, o_ref):
        x = g_ref[...] + pos_ref[...]
        tid = tid_ref[...]
        x = x + jnp.where(tid < 1, ttab_ref[0:1, :], ttab_ref[1:2, :])
        mean = jnp.mean(x, axis=-1, keepdims=True)
        xc = x - mean
        var = jnp.mean(xc * xc, axis=-1, keepdims=True)
        o_ref[...] = xc * lax.rsqrt(var + _EPS) * gam_ref[...] + bet_ref[...]

    return pl.pallas_call(
        body,
        grid=(n,),
        in_specs=[
            pl.BlockSpec((blk, hdim), lambda i: (i, 0)),
            pl.BlockSpec((blk, hdim), lambda i: (i % pos_blocks, 0)),
            pl.BlockSpec((blk, 1), lambda i: (i, 0)),
            pl.BlockSpec((8, hdim), lambda i: (0, 0)),
            pl.BlockSpec((1, hdim), lambda i: (0, 0)),
            pl.BlockSpec((1, hdim), lambda i: (0, 0)),
        ],
        out_specs=pl.BlockSpec((blk, hdim), lambda i: (i, 0)),
        out_shape=jax.ShapeDtypeStruct((tok, hdim), jnp.float32),
    )(gsum, pos_emb, tids2, tt_pad, gamma2, beta2)


def kernel(input_ids, bbox, token_type_ids, word_emb, pos_emb,
           x_emb, y_emb, h_emb, w_emb, tt_emb, ln_gamma, ln_beta):
    b, s = input_ids.shape
    hdim = word_emb.shape[1]
    tok = b * s

    ids_flat = input_ids.reshape(tok)
    bbox_t = bbox.reshape(tok, 4).T

    gsum = _sc_gather_sum(word_emb, x_emb, y_emb, h_emb, w_emb,
                          ids_flat, bbox_t)

    tids2 = token_type_ids.reshape(tok, 1)
    tt_pad = jnp.zeros((8, hdim), tt_emb.dtype).at[:2, :].set(tt_emb)
    out = _tc_finish(gsum, pos_emb, tids2, tt_pad,
                     ln_gamma.reshape(1, hdim), ln_beta.reshape(1, hdim))
    return out.reshape(b, s, hdim)


# speedup vs baseline: 2.1677x; 1.2206x over previous
import functools

import jax
import jax.numpy as jnp
from jax import lax
from jax.experimental import pallas as pl
from jax.experimental.pallas import tpu as pltpu
from jax.experimental.pallas import tpu_sc as plsc

_EPS = 1e-12
_NC, _NS = 2, 16
_NW = _NC * _NS
_LANES = 16


def _sc_gather_sum(word_emb, x_emb, y_emb, h_emb, w_emb, ids_flat, bbox_t):
    tok = ids_flat.shape[0]
    hdim = word_emb.shape[1]
    b_per_w = tok // _NW
    ch = 32
    n_chunks = b_per_w // ch
    assert tok % _NW == 0 and b_per_w % ch == 0 and hdim % _LANES == 0

    mesh = plsc.VectorSubcoreMesh(
        core_axis_name="c", subcore_axis_name="s",
        num_cores=_NC, num_subcores=_NS)

    @functools.partial(
        pl.kernel,
        out_type=jax.ShapeDtypeStruct((tok, hdim), jnp.float32),
        mesh=mesh,
        scratch_types=[
            pltpu.VMEM((ch,), jnp.int32),
            pltpu.VMEM((4, ch), jnp.int32),
            pltpu.VMEM((2, ch), jnp.int32),
            pltpu.VMEM((ch, hdim), jnp.float32),
            pltpu.VMEM((ch, hdim), jnp.float32),
            pltpu.VMEM((ch, hdim), jnp.float32),
            pltpu.SemaphoreType.DMA,
            pltpu.SemaphoreType.DMA,
            pltpu.SemaphoreType.DMA,
        ],
    )
    def k(word_hbm, x_hbm, y_hbm, h_hbm, w_hbm, ids_hbm, bbox_hbm, out_hbm,
          widx, bidx, hwidx, acc, buf_a, buf_b, sem_w, sem_a, sem_b):
        sid = lax.axis_index("s")
        wid = sid * _NC + lax.axis_index("c")

        @pl.loop(0, n_chunks)
        def _chunk(c):
            base = wid * b_per_w + c * ch
            pltpu.sync_copy(ids_hbm.at[pl.ds(base, ch)], widx)
            for j in range(4):
                pltpu.sync_copy(bbox_hbm.at[j, pl.ds(base, ch)], bidx.at[j])
            for i in range(0, ch, _LANES):
                s = pl.ds(i, _LANES)
                hwidx[0, s] = bidx[3, s] - bidx[1, s]
                hwidx[1, s] = bidx[2, s] - bidx[0, s]

            gathers = (
                (x_hbm, bidx.at[0]), (y_hbm, bidx.at[1]),
                (x_hbm, bidx.at[2]), (y_hbm, bidx.at[3]),
                (h_hbm, hwidx.at[0]), (w_hbm, hwidx.at[1]),
            )
            bufs = (buf_a, buf_b)
            sems = (sem_a, sem_b)
            cp_w = pltpu.async_copy(word_hbm.at[widx], acc, sem_w)
            cps = [None] * 6
            tab0, iref0 = gathers[0]
            cps[0] = pltpu.async_copy(tab0.at[iref0], bufs[0], sem_a)
            cp_w.wait()
            for j in range(6):
                cps[j].wait()
                if j + 1 < 6:
                    tab, iref = gathers[j + 1]
                    cps[j + 1] = pltpu.async_copy(
                        tab.at[iref], bufs[(j + 1) % 2], sems[(j + 1) % 2])
                buf = bufs[j % 2]

                @pl.loop(0, ch)
                def _row(r):
                    for i in range(0, hdim, _LANES):
                        s = pl.ds(i, _LANES)
                        plsc.addupdate(acc.at[r, s], buf[r, s])

            pltpu.sync_copy(acc, out_hbm.at[pl.ds(base, ch)])

    return k(word_emb, x_emb, y_emb, h_emb, w_emb, ids_flat, bbox_t)


def _tc_finish(gsum, pos_emb, tids2, tt_pad, gamma2, beta2):
    tok, hdim = gsum.shape
    blk = 256
    n = tok // blk
    s_len = pos_emb.shape[0]
    pos_blocks = s_len // blk

    def body(g_ref, pos_ref, tid_ref, ttab_ref, gam_ref, bet_ref, o_ref):
        x = g_ref[...] + pos_ref[...]
        tid = tid_ref[...]
        x = x + jnp.where(tid < 1, ttab_ref[0:1, :], ttab_ref[1:2, :])
        mean = jnp.mean(x, axis=-1, keepdims=True)
        xc = x - mean
        var = jnp.mean(xc * xc, axis=-1, keepdims=True)
        o_ref[...] = xc * lax.rsqrt(var + _EPS) * gam_ref[...] + bet_ref[...]

    return pl.pallas_call(
        body,
        grid=(n,),
        in_specs=[
            pl.BlockSpec((blk, hdim), lambda i: (i, 0)),
            pl.BlockSpec((blk, hdim), lambda i: (i % pos_blocks, 0)),
            pl.BlockSpec((blk, 1), lambda i: (i, 0)),
            pl.BlockSpec((8, hdim), lambda i: (0, 0)),
            pl.BlockSpec((1, hdim), lambda i: (0, 0)),
            pl.BlockSpec((1, hdim), lambda i: (0, 0)),
        ],
        out_specs=pl.BlockSpec((blk, hdim), lambda i: (i, 0)),
        out_shape=jax.ShapeDtypeStruct((tok, hdim), jnp.float32),
    )(gsum, pos_emb, tids2, tt_pad, gamma2, beta2)


def kernel(input_ids, bbox, token_type_ids, word_emb, pos_emb,
           x_emb, y_emb, h_emb, w_emb, tt_emb, ln_gamma, ln_beta):
    b, s = input_ids.shape
    hdim = word_emb.shape[1]
    tok = b * s

    ids_flat = input_ids.reshape(tok)
    bbox_t = bbox.reshape(tok, 4).T

    gsum = _sc_gather_sum(word_emb, x_emb, y_emb, h_emb, w_emb,
                          ids_flat, bbox_t)

    tids2 = token_type_ids.reshape(tok, 1)
    tt_pad = jnp.zeros((8, hdim), tt_emb.dtype).at[:2, :].set(tt_emb)
    out = _tc_finish(gsum, pos_emb, tids2, tt_pad,
                     ln_gamma.reshape(1, hdim), ln_beta.reshape(1, hdim))
    return out.reshape(b, s, hdim)
